# baseline (speedup 1.0000x reference)
import functools

import jax
import jax.numpy as jnp
from jax import lax
from jax.experimental import pallas as pl
from jax.experimental.pallas import tpu as pltpu
from jax.experimental.pallas import tpu_sc as plsc

NC = 2
NS = 16
K = 80


def _sc_rows(table, src2, dst3, npad, width, k):
    K = k
    ept = src2.shape[1]
    n_chunks = ept // K
    rows_per_tile = npad // NS
    zeros_tab = jnp.zeros((npad, width), jnp.float32)

    mesh = plsc.VectorSubcoreMesh(core_axis_name="c", subcore_axis_name="s",
                                  num_cores=NC, num_subcores=NS)

    @functools.partial(
        pl.kernel,
        out_type=jax.ShapeDtypeStruct((NC, npad, width), jnp.float32),
        mesh=mesh,
        scratch_types=(
            pltpu.VMEM((ept,), jnp.int32),
            pltpu.VMEM((n_chunks, K), jnp.int32),
            pltpu.VMEM((K, width), jnp.float32),
            pltpu.VMEM((K, width), jnp.float32),
            pltpu.VMEM_SHARED((npad, width), jnp.float32),
            pltpu.SemaphoreType.DMA,
            pltpu.SemaphoreType.DMA,
        ),
        compiler_params=pltpu.CompilerParams(
            use_tc_tiling_on_sc=(width % 128 == 0)),
    )
    def agg(table_hbm, src2_hbm, dst3_hbm, ztab_hbm, out_hbm,
            src_v, dst_v, rows0, rows1, acc_s, sem0, sem1):
        c = lax.axis_index("c")
        s = lax.axis_index("s")
        wid = c * NS + s
        base = s * rows_per_tile

        pltpu.sync_copy(src2_hbm.at[wid], src_v)
        pltpu.sync_copy(dst3_hbm.at[wid], dst_v)
        pltpu.sync_copy(ztab_hbm.at[pl.ds(base, rows_per_tile)],
                        acc_s.at[pl.ds(base, rows_per_tile)])
        plsc.subcore_barrier()

        def start_g(i, buf, sem):
            pltpu.async_copy(table_hbm.at[src_v.at[pl.ds(i * K, K)]], buf, sem)

        def wait_g(i, buf, sem):
            pltpu.make_async_copy(
                table_hbm.at[src_v.at[pl.ds(i * K, K)]], buf, sem).wait()

        def scat(i, buf):
            pltpu.sync_copy(buf, acc_s.at[dst_v.at[i]], add=True)

        start_g(0, rows0, sem0)

        @pl.loop(0, n_chunks - 1, step=2)
        def _(i):
            start_g(i + 1, rows1, sem1)
            wait_g(i, rows0, sem0)
            scat(i, rows0)
            start_g(i + 2, rows0, sem0)
            wait_g(i + 1, rows1, sem1)
            scat(i + 1, rows1)

        wait_g(n_chunks - 1, rows0, sem0)
        scat(n_chunks - 1, rows0)

        plsc.subcore_barrier()
        pltpu.sync_copy(acc_s.at[pl.ds(base, rows_per_tile)],
                        out_hbm.at[c, pl.ds(base, rows_per_tile)])

    return agg(table, src2, dst3, zeros_tab)


def _sc_deg(dst2, npad):
    ept = dst2.shape[1]

    mesh = plsc.VectorSubcoreMesh(core_axis_name="c", subcore_axis_name="s",
                                  num_cores=NC, num_subcores=NS)

    @functools.partial(
        pl.kernel,
        out_type=jax.ShapeDtypeStruct((NC * NS, npad), jnp.float32),
        mesh=mesh,
        scratch_types=(
            pltpu.VMEM((ept,), jnp.int32),
            pltpu.VMEM((npad,), jnp.float32),
        ),
        compiler_params=pltpu.CompilerParams(needs_layout_passes=False),
    )
    def degk(dst2_hbm, out_hbm, dst_v, hist_v):
        c = lax.axis_index("c")
        s = lax.axis_index("s")
        wid = c * NS + s
        zero16 = jnp.zeros((16,), jnp.float32)
        one16 = jnp.ones((16,), jnp.float32)

        pltpu.sync_copy(dst2_hbm.at[wid], dst_v)

        @pl.loop(0, npad // 16)
        def _(i):
            hist_v[pl.ds(i * 16, 16)] = zero16

        @pl.loop(0, ept // 16)
        def _(i):
            idx = dst_v[pl.ds(i * 16, 16)]
            plsc.addupdate_scatter(hist_v, [idx], one16)

        pltpu.sync_copy(hist_v, out_hbm.at[wid])

    return degk(dst2)


def _tc_layer1(x, p, hist, ws1, wn1, b1, ws2p, wn2p, b2p, block_rows):
    n, d = x.shape
    wpad = ws2p.shape[1]

    def body(x_ref, p_ref, h_ref, ws1_ref, wn1_ref, b1_ref, ws2_ref,
             wn2_ref, b2_ref, hs_ref, hp_ref, rdeg_ref):
        deg = jnp.sum(h_ref[...], axis=0, keepdims=True)
        rdeg = (1.0 / jnp.maximum(deg, 1.0)).T
        rdeg_ref[...] = rdeg
        mean = (p_ref[0] + p_ref[1]) * rdeg
        h = jnp.dot(x_ref[...], ws1_ref[...], preferred_element_type=jnp.float32)
        h += jnp.dot(mean, wn1_ref[...], preferred_element_type=jnp.float32)
        h = jnp.maximum(h + b1_ref[...], 0.0)
        hs_ref[...] = (
            jnp.dot(h, ws2_ref[...], preferred_element_type=jnp.float32)
            + b2_ref[...]
        )
        hp_ref[...] = jnp.dot(h, wn2_ref[...], preferred_element_type=jnp.float32)

    grid = (n // block_rows,)
    return pl.pallas_call(
        body,
        grid=grid,
        in_specs=[
            pl.BlockSpec((block_rows, d), lambda i: (i, 0)),
            pl.BlockSpec((NC, block_rows, d), lambda i: (0, i, 0)),
            pl.BlockSpec((NC * NS, block_rows), lambda i: (0, i)),
            pl.BlockSpec(ws1.shape, lambda i: (0, 0)),
            pl.BlockSpec(wn1.shape, lambda i: (0, 0)),
            pl.BlockSpec(b1.shape, lambda i: (0, 0)),
            pl.BlockSpec(ws2p.shape, lambda i: (0, 0)),
            pl.BlockSpec(wn2p.shape, lambda i: (0, 0)),
            pl.BlockSpec(b2p.shape, lambda i: (0, 0)),
        ],
        out_specs=[
            pl.BlockSpec((block_rows, wpad), lambda i: (i, 0)),
            pl.BlockSpec((block_rows, wpad), lambda i: (i, 0)),
            pl.BlockSpec((block_rows, 1), lambda i: (i, 0)),
        ],
        out_shape=[
            jax.ShapeDtypeStruct((n, wpad), jnp.float32),
            jax.ShapeDtypeStruct((n, wpad), jnp.float32),
            jax.ShapeDtypeStruct((n, 1), jnp.float32),
        ],
    )(x, p, hist, ws1, wn1, b1, ws2p, wn2p, b2p)


def _tc_layer2(hs, q, rdeg, block_rows):
    n, wpad = hs.shape

    def body(hs_ref, q_ref, rdeg_ref, out_ref):
        out_ref[...] = hs_ref[...] + (q_ref[0] + q_ref[1]) * rdeg_ref[...]

    grid = (n // block_rows,)
    return pl.pallas_call(
        body,
        grid=grid,
        in_specs=[
            pl.BlockSpec((block_rows, wpad), lambda i: (i, 0)),
            pl.BlockSpec((NC, block_rows, wpad), lambda i: (0, i, 0)),
            pl.BlockSpec((block_rows, 1), lambda i: (i, 0)),
        ],
        out_specs=pl.BlockSpec((block_rows, wpad), lambda i: (i, 0)),
        out_shape=jax.ShapeDtypeStruct((n, wpad), jnp.float32),
    )(hs, q, rdeg)


def kernel(features, edge_index, W_self1, W_neigh1, b1, W_self2, W_neigh2, b2):
    n, d = features.shape
    c = W_self2.shape[1]
    wpad = 48
    npad = ((n + 2047) // 2048) * 2048

    src = edge_index[0]
    dst = edge_index[1]

    ws2p = jnp.pad(W_self2, ((0, 0), (0, wpad - c)))
    wn2p = jnp.pad(W_neigh2, ((0, 0), (0, wpad - c)))
    b1r = b1.reshape(1, -1)
    b2p = jnp.pad(b2, (0, wpad - c)).reshape(1, -1)

    ept = src.shape[0] // (NC * NS)
    dst2 = dst.reshape(NC * NS, ept)

    def pad_edges(k):
        eptp = ((ept + k - 1) // k) * k
        pad = eptp - ept
        s2 = src.reshape(NC * NS, ept)
        d2 = dst2
        if pad:
            s2 = jnp.pad(s2, ((0, 0), (0, pad)))
            d2 = jnp.pad(d2, ((0, 0), (0, pad)), constant_values=npad - 1)
        return s2, d2.reshape(NC * NS, eptp // k, k)

    src2a, dst3a = pad_edges(96)
    src2b, dst3b = pad_edges(128)

    p = _sc_rows(features, src2a, dst3a, npad, d, 96)
    hist = _sc_deg(dst2, npad)
    xpad = jnp.pad(features, ((0, npad - n), (0, 0)))
    hs, hp, rdeg = _tc_layer1(xpad, p, hist, W_self1,
                              W_neigh1, b1r, ws2p, wn2p, b2p, block_rows=2048)
    q = _sc_rows(hp, src2b, dst3b, npad, wpad, 128)
    out = _tc_layer2(hs, q, rdeg, block_rows=2048)
    return out[:n, :c]


# speedup vs baseline: 1.2771x; 1.2771x over previous
import functools

import jax
import jax.numpy as jnp
from jax import lax
from jax.experimental import pallas as pl
from jax.experimental.pallas import tpu as pltpu
from jax.experimental.pallas import tpu_sc as plsc

NC = 2
NS = 16
K = 80


def _sc_rows(table, src2, dst3, npad, width, k):
    K = k
    ept = src2.shape[1]
    n_chunks = ept // K
    rows_per_tile = npad // NS
    zeros_tab = jnp.zeros((npad, width), jnp.float32)

    mesh = plsc.VectorSubcoreMesh(core_axis_name="c", subcore_axis_name="s",
                                  num_cores=NC, num_subcores=NS)

    @functools.partial(
        pl.kernel,
        out_type=jax.ShapeDtypeStruct((NC, npad, width), jnp.float32),
        mesh=mesh,
        scratch_types=(
            pltpu.VMEM((ept,), jnp.int32),
            pltpu.VMEM((n_chunks, K), jnp.int32),
            pltpu.VMEM((K, width), jnp.float32),
            pltpu.VMEM((K, width), jnp.float32),
            pltpu.VMEM_SHARED((npad, width), jnp.float32),
            pltpu.SemaphoreType.DMA,
            pltpu.SemaphoreType.DMA,
        ),
        compiler_params=pltpu.CompilerParams(
            use_tc_tiling_on_sc=(width % 128 == 0)),
    )
    def agg(table_hbm, src2_hbm, dst3_hbm, ztab_hbm, out_hbm,
            src_v, dst_v, rows0, rows1, acc_s, sem0, sem1):
        c = lax.axis_index("c")
        s = lax.axis_index("s")
        wid = c * NS + s
        base = s * rows_per_tile

        pltpu.sync_copy(src2_hbm.at[wid], src_v)
        pltpu.sync_copy(dst3_hbm.at[wid], dst_v)
        pltpu.sync_copy(ztab_hbm.at[pl.ds(base, rows_per_tile)],
                        acc_s.at[pl.ds(base, rows_per_tile)])
        plsc.subcore_barrier()

        def start_g(i, buf, sem):
            pltpu.async_copy(table_hbm.at[src_v.at[pl.ds(i * K, K)]], buf, sem)

        def wait_g(i, buf, sem):
            pltpu.make_async_copy(
                table_hbm.at[src_v.at[pl.ds(i * K, K)]], buf, sem).wait()

        def scat(i, buf):
            pltpu.sync_copy(buf, acc_s.at[dst_v.at[i]], add=True)

        start_g(0, rows0, sem0)

        @pl.loop(0, n_chunks - 1, step=2)
        def _(i):
            start_g(i + 1, rows1, sem1)
            wait_g(i, rows0, sem0)
            scat(i, rows0)
            start_g(i + 2, rows0, sem0)
            wait_g(i + 1, rows1, sem1)
            scat(i + 1, rows1)

        wait_g(n_chunks - 1, rows0, sem0)
        scat(n_chunks - 1, rows0)

        plsc.subcore_barrier()
        pltpu.sync_copy(acc_s.at[pl.ds(base, rows_per_tile)],
                        out_hbm.at[c, pl.ds(base, rows_per_tile)])

    return agg(table, src2, dst3, zeros_tab)


def _sc_deg(dst2, npad):
    ept = dst2.shape[1]

    mesh = plsc.VectorSubcoreMesh(core_axis_name="c", subcore_axis_name="s",
                                  num_cores=NC, num_subcores=NS)

    @functools.partial(
        pl.kernel,
        out_type=jax.ShapeDtypeStruct((NC * NS, npad), jnp.float32),
        mesh=mesh,
        scratch_types=(
            pltpu.VMEM((ept,), jnp.int32),
            pltpu.VMEM((npad,), jnp.float32),
        ),
        compiler_params=pltpu.CompilerParams(needs_layout_passes=False),
    )
    def degk(dst2_hbm, out_hbm, dst_v, hist_v):
        c = lax.axis_index("c")
        s = lax.axis_index("s")
        wid = c * NS + s
        zero16 = jnp.zeros((16,), jnp.float32)
        one16 = jnp.ones((16,), jnp.float32)

        pltpu.sync_copy(dst2_hbm.at[wid], dst_v)

        @pl.loop(0, npad // 16)
        def _(i):
            hist_v[pl.ds(i * 16, 16)] = zero16

        @pl.loop(0, ept // 16)
        def _(i):
            idx = dst_v[pl.ds(i * 16, 16)]
            plsc.addupdate_scatter(hist_v, [idx], one16)

        pltpu.sync_copy(hist_v, out_hbm.at[wid])

    return degk(dst2)


def _tc_layer1(x, p, hist, ws1, wn1, b1, ws2p, wn2p, b2p, block_rows):
    n, d = x.shape
    wpad = ws2p.shape[1]

    def body(x_ref, p_ref, h_ref, ws1_ref, wn1_ref, b1_ref, ws2_ref,
             wn2_ref, b2_ref, hs_ref, hp_ref, rdeg_ref):
        deg = jnp.sum(h_ref[...], axis=0, keepdims=True)
        rdeg = (1.0 / jnp.maximum(deg, 1.0)).T
        rdeg_ref[...] = rdeg
        mean = (p_ref[0] + p_ref[1]) * rdeg
        h = jnp.dot(x_ref[...], ws1_ref[...], preferred_element_type=jnp.float32)
        h += jnp.dot(mean, wn1_ref[...], preferred_element_type=jnp.float32)
        h = jnp.maximum(h + b1_ref[...], 0.0)
        hs_ref[...] = (
            jnp.dot(h, ws2_ref[...], preferred_element_type=jnp.float32)
            + b2_ref[...]
        )
        hp_ref[...] = jnp.dot(h, wn2_ref[...], preferred_element_type=jnp.float32)

    grid = (n // block_rows,)
    return pl.pallas_call(
        body,
        grid=grid,
        in_specs=[
            pl.BlockSpec((block_rows, d), lambda i: (i, 0)),
            pl.BlockSpec((NC, block_rows, d), lambda i: (0, i, 0)),
            pl.BlockSpec((NC * NS, block_rows), lambda i: (0, i)),
            pl.BlockSpec(ws1.shape, lambda i: (0, 0)),
            pl.BlockSpec(wn1.shape, lambda i: (0, 0)),
            pl.BlockSpec(b1.shape, lambda i: (0, 0)),
            pl.BlockSpec(ws2p.shape, lambda i: (0, 0)),
            pl.BlockSpec(wn2p.shape, lambda i: (0, 0)),
            pl.BlockSpec(b2p.shape, lambda i: (0, 0)),
        ],
        out_specs=[
            pl.BlockSpec((block_rows, wpad), lambda i: (i, 0)),
            pl.BlockSpec((block_rows, wpad), lambda i: (i, 0)),
            pl.BlockSpec((block_rows, 1), lambda i: (i, 0)),
        ],
        out_shape=[
            jax.ShapeDtypeStruct((n, wpad), jnp.float32),
            jax.ShapeDtypeStruct((n, wpad), jnp.float32),
            jax.ShapeDtypeStruct((n, 1), jnp.float32),
        ],
    )(x, p, hist, ws1, wn1, b1, ws2p, wn2p, b2p)


def _tc_layer2(hs, q, rdeg, block_rows):
    n, wpad = hs.shape

    def body(hs_ref, q_ref, rdeg_ref, out_ref):
        out_ref[...] = hs_ref[...] + (q_ref[0] + q_ref[1]) * rdeg_ref[...]

    grid = (n // block_rows,)
    return pl.pallas_call(
        body,
        grid=grid,
        in_specs=[
            pl.BlockSpec((block_rows, wpad), lambda i: (i, 0)),
            pl.BlockSpec((NC, block_rows, wpad), lambda i: (0, i, 0)),
            pl.BlockSpec((block_rows, 1), lambda i: (i, 0)),
        ],
        out_specs=pl.BlockSpec((block_rows, wpad), lambda i: (i, 0)),
        out_shape=jax.ShapeDtypeStruct((n, wpad), jnp.float32),
    )(hs, q, rdeg)


def kernel(features, edge_index, W_self1, W_neigh1, b1, W_self2, W_neigh2, b2):
    n, d = features.shape
    c = W_self2.shape[1]
    wpad = 48
    npad = ((n + 2047) // 2048) * 2048

    src = edge_index[0]
    dst = edge_index[1]

    ws2p = jnp.pad(W_self2, ((0, 0), (0, wpad - c)))
    wn2p = jnp.pad(W_neigh2, ((0, 0), (0, wpad - c)))
    b1r = b1.reshape(1, -1)
    b2p = jnp.pad(b2, (0, wpad - c)).reshape(1, -1)

    ept = src.shape[0] // (NC * NS)
    dst2 = dst.reshape(NC * NS, ept)

    def pad_edges(k):
        eptp = ((ept + k - 1) // k) * k
        pad = eptp - ept
        s2 = src.reshape(NC * NS, ept)
        d2 = dst2
        if pad:
            s2 = jnp.pad(s2, ((0, 0), (0, pad)))
            d2 = jnp.pad(d2, ((0, 0), (0, pad)), constant_values=npad - 1)
        return s2, d2.reshape(NC * NS, eptp // k, k)

    src2a, dst3a = pad_edges(80)
    src2b, dst3b = pad_edges(128)

    p = _sc_rows(features, src2a, dst3a, npad, d, 80)
    hist = _sc_deg(dst2, npad)
    xpad = jnp.pad(features, ((0, npad - n), (0, 0)))
    hs, hp, rdeg = _tc_layer1(xpad, p, hist, W_self1,
                              W_neigh1, b1r, ws2p, wn2p, b2p, block_rows=2048)
    q = _sc_rows(hp, src2b, dst3b, npad, wpad, 128)
    out = _tc_layer2(hs, q, rdeg, block_rows=2048)
    return out[:n, :c]


# speedup vs baseline: 1.4315x; 1.1209x over previous
import functools

import jax
import jax.numpy as jnp
from jax import lax
from jax.experimental import pallas as pl
from jax.experimental.pallas import tpu as pltpu
from jax.experimental.pallas import tpu_sc as plsc

NC = 2
NS = 16
K = 80


def _sc_rows(table, src2, dst3, npad, width, k):
    K = k
    ept = src2.shape[1]
    n_chunks = ept // K
    rows_per_tile = npad // NS
    zeros_tab = jnp.zeros((npad, width), jnp.float32)

    mesh = plsc.VectorSubcoreMesh(core_axis_name="c", subcore_axis_name="s",
                                  num_cores=NC, num_subcores=NS)

    @functools.partial(
        pl.kernel,
        out_type=jax.ShapeDtypeStruct((NC, npad, width), jnp.float32),
        mesh=mesh,
        scratch_types=(
            pltpu.VMEM((ept,), jnp.int32),
            pltpu.VMEM((n_chunks, K), jnp.int32),
            pltpu.VMEM((K, width), jnp.float32),
            pltpu.VMEM((K, width), jnp.float32),
            pltpu.VMEM_SHARED((npad, width), jnp.float32),
            pltpu.SemaphoreType.DMA,
            pltpu.SemaphoreType.DMA,
        ),
        compiler_params=pltpu.CompilerParams(
            use_tc_tiling_on_sc=(width % 128 == 0)),
    )
    def agg(table_hbm, src2_hbm, dst3_hbm, ztab_hbm, out_hbm,
            src_v, dst_v, rows0, rows1, acc_s, sem0, sem1):
        c = lax.axis_index("c")
        s = lax.axis_index("s")
        wid = c * NS + s
        base = s * rows_per_tile

        pltpu.sync_copy(src2_hbm.at[wid], src_v)
        pltpu.sync_copy(dst3_hbm.at[wid], dst_v)
        pltpu.sync_copy(ztab_hbm.at[pl.ds(base, rows_per_tile)],
                        acc_s.at[pl.ds(base, rows_per_tile)])
        plsc.subcore_barrier()

        def start_g(i, buf, sem):
            pltpu.async_copy(table_hbm.at[src_v.at[pl.ds(i * K, K)]], buf, sem)

        def wait_g(i, buf, sem):
            pltpu.make_async_copy(
                table_hbm.at[src_v.at[pl.ds(i * K, K)]], buf, sem).wait()

        def scat(i, buf):
            pltpu.sync_copy(buf, acc_s.at[dst_v.at[i]], add=True)

        start_g(0, rows0, sem0)

        @pl.loop(0, n_chunks - 1, step=2)
        def _(i):
            start_g(i + 1, rows1, sem1)
            wait_g(i, rows0, sem0)
            scat(i, rows0)
            start_g(i + 2, rows0, sem0)
            wait_g(i + 1, rows1, sem1)
            scat(i + 1, rows1)

        wait_g(n_chunks - 1, rows0, sem0)
        scat(n_chunks - 1, rows0)

        plsc.subcore_barrier()
        pltpu.sync_copy(acc_s.at[pl.ds(base, rows_per_tile)],
                        out_hbm.at[c, pl.ds(base, rows_per_tile)])

    return agg(table, src2, dst3, zeros_tab)


def _sc_deg(dst2, npad):
    ept = dst2.shape[1]

    mesh = plsc.VectorSubcoreMesh(core_axis_name="c", subcore_axis_name="s",
                                  num_cores=NC, num_subcores=NS)

    @functools.partial(
        pl.kernel,
        out_type=jax.ShapeDtypeStruct((NC * NS, npad), jnp.float32),
        mesh=mesh,
        scratch_types=(
            pltpu.VMEM((ept,), jnp.int32),
            pltpu.VMEM((npad,), jnp.float32),
        ),
        compiler_params=pltpu.CompilerParams(needs_layout_passes=False),
    )
    def degk(dst2_hbm, out_hbm, dst_v, hist_v):
        c = lax.axis_index("c")
        s = lax.axis_index("s")
        wid = c * NS + s
        zero16 = jnp.zeros((16,), jnp.float32)
        one16 = jnp.ones((16,), jnp.float32)

        pltpu.sync_copy(dst2_hbm.at[wid], dst_v)

        @pl.loop(0, npad // 16)
        def _(i):
            hist_v[pl.ds(i * 16, 16)] = zero16

        @pl.loop(0, ept // 16)
        def _(i):
            idx = dst_v[pl.ds(i * 16, 16)]
            plsc.addupdate_scatter(hist_v, [idx], one16)

        pltpu.sync_copy(hist_v, out_hbm.at[wid])

    return degk(dst2)


def _tc_layer1(x, p, hist, ws1, wn1, b1, ws2p, wn2p, b2p, block_rows):
    n, d = x.shape
    wpad = ws2p.shape[1]

    def body(x_ref, p_ref, h_ref, ws1_ref, wn1_ref, b1_ref, ws2_ref,
             wn2_ref, b2_ref, hs_ref, hp_ref, rdeg_ref):
        deg = jnp.sum(h_ref[...], axis=0, keepdims=True)
        rdeg = (1.0 / jnp.maximum(deg, 1.0)).T
        rdeg_ref[...] = rdeg
        mean = (p_ref[0] + p_ref[1]) * rdeg
        h = jnp.dot(x_ref[...], ws1_ref[...], preferred_element_type=jnp.float32)
        h += jnp.dot(mean, wn1_ref[...], preferred_element_type=jnp.float32)
        h = jnp.maximum(h + b1_ref[...], 0.0)
        hs_ref[...] = (
            jnp.dot(h, ws2_ref[...], preferred_element_type=jnp.float32)
            + b2_ref[...]
        )
        hp_ref[...] = jnp.dot(h, wn2_ref[...], preferred_element_type=jnp.float32)

    grid = (n // block_rows,)
    return pl.pallas_call(
        body,
        grid=grid,
        in_specs=[
            pl.BlockSpec((block_rows, d), lambda i: (i, 0)),
            pl.BlockSpec((NC, block_rows, d), lambda i: (0, i, 0)),
            pl.BlockSpec((NC * NS, block_rows), lambda i: (0, i)),
            pl.BlockSpec(ws1.shape, lambda i: (0, 0)),
            pl.BlockSpec(wn1.shape, lambda i: (0, 0)),
            pl.BlockSpec(b1.shape, lambda i: (0, 0)),
            pl.BlockSpec(ws2p.shape, lambda i: (0, 0)),
            pl.BlockSpec(wn2p.shape, lambda i: (0, 0)),
            pl.BlockSpec(b2p.shape, lambda i: (0, 0)),
        ],
        out_specs=[
            pl.BlockSpec((block_rows, wpad), lambda i: (i, 0)),
            pl.BlockSpec((block_rows, wpad), lambda i: (i, 0)),
            pl.BlockSpec((block_rows, 1), lambda i: (i, 0)),
        ],
        out_shape=[
            jax.ShapeDtypeStruct((n, wpad), jnp.float32),
            jax.ShapeDtypeStruct((n, wpad), jnp.float32),
            jax.ShapeDtypeStruct((n, 1), jnp.float32),
        ],
    )(x, p, hist, ws1, wn1, b1, ws2p, wn2p, b2p)


def _tc_layer2(hs, q, rdeg, block_rows):
    n, wpad = hs.shape

    def body(hs_ref, q_ref, rdeg_ref, out_ref):
        out_ref[...] = hs_ref[...] + (q_ref[0] + q_ref[1]) * rdeg_ref[...]

    grid = (n // block_rows,)
    return pl.pallas_call(
        body,
        grid=grid,
        in_specs=[
            pl.BlockSpec((block_rows, wpad), lambda i: (i, 0)),
            pl.BlockSpec((NC, block_rows, wpad), lambda i: (0, i, 0)),
            pl.BlockSpec((block_rows, 1), lambda i: (i, 0)),
        ],
        out_specs=pl.BlockSpec((block_rows, wpad), lambda i: (i, 0)),
        out_shape=jax.ShapeDtypeStruct((n, wpad), jnp.float32),
    )(hs, q, rdeg)


def kernel(features, edge_index, W_self1, W_neigh1, b1, W_self2, W_neigh2, b2):
    n, d = features.shape
    c = W_self2.shape[1]
    wpad = 48
    npad = ((n + 2047) // 2048) * 2048

    src = edge_index[0]
    dst = edge_index[1]

    ws2p = jnp.pad(W_self2, ((0, 0), (0, wpad - c)))
    wn2p = jnp.pad(W_neigh2, ((0, 0), (0, wpad - c)))
    b1r = b1.reshape(1, -1)
    b2p = jnp.pad(b2, (0, wpad - c)).reshape(1, -1)

    ept = src.shape[0] // (NC * NS)
    dst2 = dst.reshape(NC * NS, ept)

    def pad_edges(k):
        eptp = ((ept + k - 1) // k) * k
        pad = eptp - ept
        s2 = src.reshape(NC * NS, ept)
        d2 = dst2
        if pad:
            s2 = jnp.pad(s2, ((0, 0), (0, pad)))
            d2 = jnp.pad(d2, ((0, 0), (0, pad)), constant_values=npad - 1)
        return s2, d2.reshape(NC * NS, eptp // k, k)

    src2a, dst3a = pad_edges(80)
    src2b, dst3b = pad_edges(80)

    p = _sc_rows(features, src2a, dst3a, npad, d, 80)
    hist = _sc_deg(dst2, npad)
    xpad = jnp.pad(features, ((0, npad - n), (0, 0)))
    hs, hp, rdeg = _tc_layer1(xpad, p, hist, W_self1,
                              W_neigh1, b1r, ws2p, wn2p, b2p, block_rows=2048)
    q = _sc_rows(hp, src2b, dst3b, npad, wpad, 80)
    out = _tc_layer2(hs, q, rdeg, block_rows=2048)
    return out[:n, :c]
